# baseline (speedup 1.0000x reference)
import functools

import jax
import jax.numpy as jnp
from jax import lax
from jax.experimental import pallas as pl
from jax.experimental.pallas import tpu as pltpu
from jax.experimental.pallas import tpu_sc as plsc

_NC, _NS = 2, 16
_NW = _NC * _NS


def _gather_rows(table, idx, chunk):
    V, D = table.shape
    B = idx.shape[0]
    assert B % (_NW * chunk) == 0
    b_per_w = B // _NW
    n_chunks = b_per_w // chunk
    mesh = plsc.VectorSubcoreMesh(core_axis_name="c", subcore_axis_name="s")

    @functools.partial(
        pl.kernel,
        mesh=mesh,
        out_type=jax.ShapeDtypeStruct((B, D), jnp.float32),
        scratch_types=[
            pltpu.VMEM((b_per_w,), jnp.int32),
            pltpu.VMEM((chunk, D), jnp.float32),
            pltpu.SemaphoreType.DMA,
        ],
    )
    def k(table_hbm, idx_hbm, out_hbm, idx_v, rows_v, sem):
        wid = lax.axis_index("s") * _NC + lax.axis_index("c")
        base = wid * b_per_w
        pltpu.sync_copy(idx_hbm.at[pl.ds(base, b_per_w)], idx_v)

        @pl.loop(0, n_chunks)
        def _(ci):
            off = ci * chunk
            pltpu.async_copy(
                table_hbm.at[idx_v.at[pl.ds(off, chunk)]], rows_v, sem
            ).wait()
            pltpu.sync_copy(rows_v, out_hbm.at[pl.ds(base + off, chunk)])

    return k(table, idx)


def kernel(input_features, project_map):
    B, C, H, W = input_features.shape
    Ho, Wo, _ = project_map.shape
    rows = project_map[:, :, 0].astype(jnp.int32)
    cols = project_map[:, :, 1].astype(jnp.int32)
    idx = (rows * W + cols).reshape(-1)
    table = input_features.transpose(2, 3, 0, 1).reshape(H * W, B * C)
    out_t = _gather_rows(table, idx, chunk=128)
    return out_t.reshape(Ho, Wo, B, C).transpose(2, 3, 0, 1)
